# Initial kernel scaffold; baseline (speedup 1.0000x reference)
import jax
import jax.numpy as jnp
from jax.experimental import pallas as pl


def kernel(z_e, embedding_weight):
    raise NotImplementedError("write your pallas kernel here")


# speedup vs baseline: 1.0460x; 1.0460x over previous
import jax
import jax.numpy as jnp
from jax.experimental import pallas as pl
from jax.experimental.pallas import tpu as pltpu

_K = 1024
_TOK_BLOCK = 1024


def _vq_kernel(z_ref, e_ref, zq_ref, sse_ref):
    z = z_ref[...]
    e = e_ref[...]
    a = jnp.sum(z * z, axis=1, keepdims=True)
    m = jax.lax.dot_general(z, e, (((1,), (1,)), ((), ())))
    b = jnp.sum(e * e, axis=1, keepdims=True).T
    dists = a - 2.0 * m + b
    mins = jnp.min(dists, axis=1, keepdims=True)
    ks = jax.lax.broadcasted_iota(jnp.int32, dists.shape, 1)
    idx = jnp.min(jnp.where(dists == mins, ks, _K), axis=1)
    onehot = (ks == idx[:, None]).astype(jnp.float32)
    zq = jax.lax.dot_general(onehot, e, (((1,), (0,)), ((), ())),
                             precision=jax.lax.Precision.HIGHEST)
    zq_ref[...] = zq
    dif = zq - z
    blk = jnp.sum(dif * dif).reshape(1, 1)

    @pl.when(pl.program_id(0) == 0)
    def _init():
        sse_ref[...] = jnp.zeros((1, 1), jnp.float32)

    sse_ref[...] += blk


def kernel(z_e, embedding_weight):
    B, D, H, W = z_e.shape
    N = B * H * W
    z_flat = jnp.transpose(z_e, (0, 2, 3, 1)).reshape(N, D)
    zq_flat, sse = pl.pallas_call(
        _vq_kernel,
        grid=(N // _TOK_BLOCK,),
        in_specs=[
            pl.BlockSpec((_TOK_BLOCK, D), lambda i: (i, 0)),
            pl.BlockSpec((_K, D), lambda i: (0, 0)),
        ],
        out_specs=[
            pl.BlockSpec((_TOK_BLOCK, D), lambda i: (i, 0)),
            pl.BlockSpec((1, 1), lambda i: (0, 0)),
        ],
        out_shape=[
            jax.ShapeDtypeStruct((N, D), jnp.float32),
            jax.ShapeDtypeStruct((1, 1), jnp.float32),
        ],
    )(z_flat, embedding_weight)
    inv = 1.0 / (N * D)
    codebook_loss = (sse[0, 0] * inv).astype(jnp.float32)
    commitment_loss = (sse[0, 0] * (0.25 * inv)).astype(jnp.float32)
    z_q = jnp.transpose(zq_flat.reshape(B, H, W, D), (0, 3, 1, 2))
    return z_q, codebook_loss, commitment_loss


# speedup vs baseline: 1.5842x; 1.5146x over previous
import functools

import jax
import jax.numpy as jnp
from jax import lax
from jax.experimental import pallas as pl
from jax.experimental.pallas import tpu as pltpu
from jax.experimental.pallas import tpu_sc as plsc

_K = 1024
_TOK_BLOCK = 1024


def _dist_kernel(z_ref, e_ref, idx_ref, sse_ref):
    z = z_ref[...]
    e = e_ref[...]
    a = jnp.sum(z * z, axis=1, keepdims=True)
    m = jax.lax.dot_general(z, e, (((1,), (1,)), ((), ())))
    b = jnp.sum(e * e, axis=1, keepdims=True).T
    dists = a - 2.0 * m + b
    mins = jnp.min(dists, axis=1, keepdims=True)
    ks = jax.lax.broadcasted_iota(jnp.int32, dists.shape, 1)
    idx = jnp.min(jnp.where(dists == mins, ks, _K), axis=1)
    idx_ref[...] = idx.reshape(1, 8, _TOK_BLOCK // 8)
    blk = jnp.sum(mins).reshape(1, 1)

    @pl.when(pl.program_id(0) == 0)
    def _init():
        sse_ref[...] = jnp.zeros((1, 1), jnp.float32)

    sse_ref[...] += blk


def _make_sc_gather(n_tok, d):
    info = plsc.get_sparse_core_info()
    nw = info.num_cores * info.num_subcores
    b_per_w = n_tok // nw
    mesh = plsc.VectorSubcoreMesh(core_axis_name="c", subcore_axis_name="s")

    @functools.partial(
        pl.kernel, mesh=mesh,
        out_type=jax.ShapeDtypeStruct((n_tok, d), jnp.float32),
        compiler_params=pltpu.CompilerParams(use_tc_tiling_on_sc=False),
        scratch_types=[
            pltpu.VMEM((b_per_w,), jnp.int32),
            pltpu.VMEM((b_per_w, d), jnp.float32),
            pltpu.SemaphoreType.DMA,
        ],
    )
    def _gather(table_hbm, idx_hbm, out_hbm, idx_v, rows_v, sem):
        wid = lax.axis_index("s") * info.num_cores + lax.axis_index("c")
        base = wid * b_per_w
        pltpu.sync_copy(idx_hbm.at[pl.ds(base, b_per_w)], idx_v)
        pltpu.async_copy(table_hbm.at[idx_v], rows_v, sem).wait()
        pltpu.sync_copy(rows_v, out_hbm.at[pl.ds(base, b_per_w)])

    return _gather


def kernel(z_e, embedding_weight):
    B, D, H, W = z_e.shape
    N = B * H * W
    z_flat = jnp.transpose(z_e, (0, 2, 3, 1)).reshape(N, D)
    nblk = N // _TOK_BLOCK
    idx3, sse = pl.pallas_call(
        _dist_kernel,
        grid=(nblk,),
        in_specs=[
            pl.BlockSpec((_TOK_BLOCK, D), lambda i: (i, 0)),
            pl.BlockSpec((_K, D), lambda i: (0, 0)),
        ],
        out_specs=[
            pl.BlockSpec((1, 8, _TOK_BLOCK // 8), lambda i: (i, 0, 0)),
            pl.BlockSpec((1, 1), lambda i: (0, 0)),
        ],
        out_shape=[
            jax.ShapeDtypeStruct((nblk, 8, _TOK_BLOCK // 8), jnp.int32),
            jax.ShapeDtypeStruct((1, 1), jnp.float32),
        ],
    )(z_flat, embedding_weight)
    idx = idx3.reshape(N)
    zq_flat = _make_sc_gather(N, D)(embedding_weight, idx)
    inv = 1.0 / (N * D)
    codebook_loss = (sse[0, 0] * inv).astype(jnp.float32)
    commitment_loss = (sse[0, 0] * (0.25 * inv)).astype(jnp.float32)
    z_q = jnp.transpose(zq_flat.reshape(B, H, W, D), (0, 3, 1, 2))
    return z_q, codebook_loss, commitment_loss
